# baseline (speedup 1.0000x reference)
import functools

import jax
import jax.numpy as jnp
from jax import lax
from jax.experimental import pallas as pl
from jax.experimental.pallas import tpu as pltpu
from jax.experimental.pallas import tpu_sc as plsc

N_TOKEN = 1000000
CUTOFF = 100000
D_EMBED = 128
D1 = 32
B_TOK = 1024 * 200

NC, NS = 2, 16
NW = NC * NS
BPW = B_TOK // NW
CH = 128
NCHUNK = BPW // CH

_SC_MESH = plsc.VectorSubcoreMesh(core_axis_name="c", subcore_axis_name="s")


@functools.partial(
    pl.kernel,
    out_type=(
        jax.ShapeDtypeStruct((B_TOK, D_EMBED), jnp.float32),
        jax.ShapeDtypeStruct((B_TOK, D_EMBED), jnp.float32),
    ),
    mesh=_SC_MESH,
    scratch_types=[
        pltpu.VMEM((2 * CH,), jnp.int32),
        pltpu.VMEM((2 * CH,), jnp.int32),
        pltpu.VMEM((2 * CH,), jnp.int32),
        pltpu.VMEM((2, CH, D_EMBED), jnp.float32),
        pltpu.VMEM((2, CH, D_EMBED), jnp.float32),
        pltpu.SemaphoreType.DMA,
        pltpu.SemaphoreType.DMA,
        pltpu.SemaphoreType.DMA,
        pltpu.SemaphoreType.DMA,
    ],
)
def _sc_gather(idx_hbm, emb0_hbm, emb1r_hbm, g0_hbm, gs_hbm,
               idxr_v, idx0_v, idxs_v, rows0_v, rowss_v, sa0, sa1, sb0, sb1):
    wid = lax.axis_index("s") * NC + lax.axis_index("c")
    base = wid * BPW
    sems0 = (sa0, sb0)
    sems1 = (sa1, sb1)

    def fire(i, sl):
        off = base + i * CH
        pltpu.sync_copy(idx_hbm.at[pl.ds(off, CH)],
                        idxr_v.at[pl.ds(sl * CH, CH)])
        for k in range(CH // 16):
            o16 = sl * CH + k * 16
            v = idxr_v[pl.ds(o16, 16)]
            m = v < CUTOFF
            idx0_v[pl.ds(o16, 16)] = jnp.where(m, v, v & 0xFFFF)
            idxs_v[pl.ds(o16, 16)] = jnp.where(
                m, v, lax.shift_right_logical(v - CUTOFF, 2))
        pltpu.async_copy(emb0_hbm.at[idx0_v.at[pl.ds(sl * CH, CH)]],
                         rows0_v.at[sl], sems0[sl])
        pltpu.async_copy(emb1r_hbm.at[idxs_v.at[pl.ds(sl * CH, CH)]],
                         rowss_v.at[sl], sems1[sl])

    def drain(i, sl):
        off = base + i * CH
        pltpu.make_async_copy(emb0_hbm.at[idx0_v.at[pl.ds(sl * CH, CH)]],
                              rows0_v.at[sl], sems0[sl]).wait()
        pltpu.sync_copy(rows0_v.at[sl], g0_hbm.at[pl.ds(off, CH)])
        pltpu.make_async_copy(emb1r_hbm.at[idxs_v.at[pl.ds(sl * CH, CH)]],
                              rowss_v.at[sl], sems1[sl]).wait()
        pltpu.sync_copy(rowss_v.at[sl], gs_hbm.at[pl.ds(off, CH)])

    fire(0, 0)

    def step(s, c):
        for par in range(2):
            i = 2 * s + par

            @pl.when(i + 1 < NCHUNK)
            def _():
                fire(i + 1, 1 - par)

            drain(i, par)
        return c

    lax.fori_loop(0, NCHUNK // 2, step, 0)


_RB = 1024
_NSR = (N_TOKEN - CUTOFF) // 4


def _relayout_body(p2_ref, o_ref):
    x = p2_ref[...]
    x3 = x.reshape(32, _RB, 4)
    o_ref[...] = x3.transpose(1, 2, 0).reshape(_RB, 128)


def _relayout(emb1):
    p2 = emb1.T
    grid = (_NSR + _RB - 1) // _RB
    return pl.pallas_call(
        _relayout_body,
        grid=(grid,),
        in_specs=[pl.BlockSpec((32, 4 * _RB), lambda i: (0, i))],
        out_specs=pl.BlockSpec((_RB, D_EMBED), lambda i: (i, 0)),
        out_shape=jax.ShapeDtypeStruct((_NSR, D_EMBED), jnp.float32),
    )(p2)


_TB = 4096


def _tc_body(m_ref, g0_ref, gs_ref, p0_ref, w1_ref, o_ref):
    dn = (((1,), (1,)), ((), ()))
    a = lax.dot_general(g0_ref[...], p0_ref[...], dn,
                        preferred_element_type=jnp.float32)
    qv = lax.rem(m_ref[...] - CUTOFF, 4)
    lane_grp = lax.broadcasted_iota(jnp.int32, (_TB, D_EMBED), 1) // D1
    gs_m = jnp.where(lane_grp == qv, gs_ref[...], 0.0)
    dnw = (((1,), (0,)), ((), ()))
    b = lax.dot_general(gs_m, w1_ref[...], dnw,
                        preferred_element_type=jnp.float32)
    scale = float(D_EMBED) ** 0.5
    o_ref[...] = jnp.where(m_ref[...] < CUTOFF, a, b) * scale


def _tc_project(idx2d, g0, gs, proj0, w1tile):
    return pl.pallas_call(
        _tc_body,
        grid=(B_TOK // _TB,),
        in_specs=[
            pl.BlockSpec((_TB, 1), lambda i: (i, 0)),
            pl.BlockSpec((_TB, D_EMBED), lambda i: (i, 0)),
            pl.BlockSpec((_TB, D_EMBED), lambda i: (i, 0)),
            pl.BlockSpec((D_EMBED, D_EMBED), lambda i: (0, 0)),
            pl.BlockSpec((D_EMBED, D_EMBED), lambda i: (0, 0)),
        ],
        out_specs=pl.BlockSpec((_TB, D_EMBED), lambda i: (i, 0)),
        out_shape=jax.ShapeDtypeStruct((B_TOK, D_EMBED), jnp.float32),
    )(idx2d, g0, gs, proj0, w1tile)


def kernel(inp, emb0, proj0, emb1, proj1):
    idx = inp.reshape(-1).astype(jnp.int32)
    emb1r = _relayout(emb1)
    w1tile = jnp.tile(proj1.T, (4, 1))
    g0, gs = _sc_gather(idx, emb0, emb1r)
    out = _tc_project(idx.reshape(B_TOK, 1), g0, gs, proj0, w1tile)
    return out.reshape(inp.shape + (D_EMBED,))


# speedup vs baseline: 3.5155x; 3.5155x over previous
import functools

import jax
import jax.numpy as jnp
from jax import lax
from jax.experimental import pallas as pl
from jax.experimental.pallas import tpu as pltpu
from jax.experimental.pallas import tpu_sc as plsc

N_TOKEN = 1000000
CUTOFF = 100000
D_EMBED = 128
D1 = 32
B_TOK = 1024 * 200

NC, NS = 2, 16
NW = NC * NS
BPW = B_TOK // NW
CH = 128
NCHUNK = BPW // CH

_SC_MESH = plsc.VectorSubcoreMesh(core_axis_name="c", subcore_axis_name="s")


@functools.partial(
    pl.kernel,
    out_type=(
        jax.ShapeDtypeStruct((B_TOK, D_EMBED), jnp.float32),
        jax.ShapeDtypeStruct((B_TOK, D_EMBED), jnp.float32),
    ),
    mesh=_SC_MESH,
    scratch_types=[
        pltpu.VMEM((2 * CH,), jnp.int32),
        pltpu.VMEM((2 * CH,), jnp.int32),
        pltpu.VMEM((2 * CH,), jnp.int32),
        pltpu.VMEM((2, CH, D_EMBED), jnp.float32),
        pltpu.VMEM((2, CH, D_EMBED), jnp.float32),
        pltpu.SemaphoreType.DMA,
        pltpu.SemaphoreType.DMA,
        pltpu.SemaphoreType.DMA,
        pltpu.SemaphoreType.DMA,
    ],
)
def _sc_gather(idx_hbm, emb0_hbm, emb1r_hbm, g0_hbm, gs_hbm,
               idxr_v, idx0_v, idxs_v, rows0_v, rowss_v, sa0, sa1, sb0, sb1):
    wid = lax.axis_index("s") * NC + lax.axis_index("c")
    base = wid * BPW
    sems0 = (sa0, sb0)
    sems1 = (sa1, sb1)

    def fire(i, sl):
        off = base + i * CH
        pltpu.sync_copy(idx_hbm.at[pl.ds(off, CH)],
                        idxr_v.at[pl.ds(sl * CH, CH)])
        for k in range(CH // 16):
            o16 = sl * CH + k * 16
            v = idxr_v[pl.ds(o16, 16)]
            m = v < CUTOFF
            idx0_v[pl.ds(o16, 16)] = jnp.where(m, v, v & 0xFFFF)
            idxs_v[pl.ds(o16, 16)] = jnp.where(
                m, v, lax.shift_right_logical(v - CUTOFF, 2))
        pltpu.async_copy(emb0_hbm.at[idx0_v.at[pl.ds(sl * CH, CH)]],
                         rows0_v.at[sl], sems0[sl])
        pltpu.async_copy(emb1r_hbm.at[idxs_v.at[pl.ds(sl * CH, CH)]],
                         rowss_v.at[sl], sems1[sl])

    def drain(i, sl):
        off = base + i * CH
        pltpu.make_async_copy(emb0_hbm.at[idx0_v.at[pl.ds(sl * CH, CH)]],
                              rows0_v.at[sl], sems0[sl]).wait()
        pltpu.sync_copy(rows0_v.at[sl], g0_hbm.at[pl.ds(off, CH)])
        pltpu.make_async_copy(emb1r_hbm.at[idxs_v.at[pl.ds(sl * CH, CH)]],
                              rowss_v.at[sl], sems1[sl]).wait()
        pltpu.sync_copy(rowss_v.at[sl], gs_hbm.at[pl.ds(off, CH)])

    fire(0, 0)

    def step(s, c):
        for par in range(2):
            i = 2 * s + par

            @pl.when(i + 1 < NCHUNK)
            def _():
                fire(i + 1, 1 - par)

            drain(i, par)
        return c

    lax.fori_loop(0, NCHUNK // 2, step, 0)


_TB = 4096


def _tc_body(m_ref, g0_ref, gs_ref, p0_ref, w1_ref, o_ref):
    dn = (((1,), (1,)), ((), ()))
    a = lax.dot_general(g0_ref[...], p0_ref[...], dn,
                        preferred_element_type=jnp.float32)
    qv = lax.rem(m_ref[...] - CUTOFF, 4)
    lane_grp = lax.broadcasted_iota(jnp.int32, (_TB, D_EMBED), 1) // D1
    gs_m = jnp.where(lane_grp == qv, gs_ref[...], 0.0)
    dnw = (((1,), (0,)), ((), ()))
    b = lax.dot_general(gs_m, w1_ref[...], dnw,
                        preferred_element_type=jnp.float32)
    scale = float(D_EMBED) ** 0.5
    o_ref[...] = jnp.where(m_ref[...] < CUTOFF, a, b) * scale


def _tc_project(idx2d, g0, gs, proj0, w1tile):
    return pl.pallas_call(
        _tc_body,
        grid=(B_TOK // _TB,),
        in_specs=[
            pl.BlockSpec((_TB, 1), lambda i: (i, 0)),
            pl.BlockSpec((_TB, D_EMBED), lambda i: (i, 0)),
            pl.BlockSpec((_TB, D_EMBED), lambda i: (i, 0)),
            pl.BlockSpec((D_EMBED, D_EMBED), lambda i: (0, 0)),
            pl.BlockSpec((D_EMBED, D_EMBED), lambda i: (0, 0)),
        ],
        out_specs=pl.BlockSpec((_TB, D_EMBED), lambda i: (i, 0)),
        out_shape=jax.ShapeDtypeStruct((B_TOK, D_EMBED), jnp.float32),
    )(idx2d, g0, gs, proj0, w1tile)


def kernel(inp, emb0, proj0, emb1, proj1):
    idx = inp.reshape(-1).astype(jnp.int32)
    emb1r = emb1.reshape((N_TOKEN - CUTOFF) // 4, D_EMBED)
    w1tile = jnp.tile(proj1.T, (4, 1))
    g0, gs = _sc_gather(idx, emb0, emb1r)
    out = _tc_project(idx.reshape(B_TOK, 1), g0, gs, proj0, w1tile)
    return out.reshape(inp.shape + (D_EMBED,))
